# baseline (speedup 1.0000x reference)
import functools

import jax
import jax.numpy as jnp
from jax import lax
from jax.experimental import pallas as pl
from jax.experimental.pallas import tpu as pltpu
from jax.experimental.pallas import tpu_sc as plsc


def _sc_gather(artist_ids, genre_ids, artist_table, genre_table):
    B = artist_ids.shape[0]
    E = artist_table.shape[1]
    info = plsc.get_sparse_core_info()
    NC, NS = info.num_cores, info.num_subcores
    NW = NC * NS
    b_per_w = B // NW
    mesh = plsc.VectorSubcoreMesh(core_axis_name="c", subcore_axis_name="s")

    @functools.partial(
        pl.kernel,
        mesh=mesh,
        compiler_params=pltpu.CompilerParams(use_tc_tiling_on_sc=False),
        out_type=[
            jax.ShapeDtypeStruct((B, E), jnp.float32),
            jax.ShapeDtypeStruct((B, E), jnp.float32),
        ],
        scratch_types=[
            pltpu.VMEM((b_per_w,), jnp.int32),
            pltpu.VMEM((b_per_w, E), jnp.float32),
            pltpu.VMEM((b_per_w,), jnp.int32),
            pltpu.VMEM((b_per_w, E), jnp.float32),
            pltpu.SemaphoreType.DMA,
            pltpu.SemaphoreType.DMA,
        ],
    )
    def gather_k(aid_hbm, gid_hbm, atab_hbm, gtab_hbm, aout_hbm, gout_hbm,
                 aidx_v, arows_v, gidx_v, grows_v, sem_a, sem_g):
        wid = lax.axis_index("s") * NC + lax.axis_index("c")
        base = wid * b_per_w
        pltpu.sync_copy(aid_hbm.at[pl.ds(base, b_per_w)], aidx_v)
        pltpu.sync_copy(gid_hbm.at[pl.ds(base, b_per_w)], gidx_v)
        ca = pltpu.async_copy(atab_hbm.at[aidx_v], arows_v, sem_a)
        cg = pltpu.async_copy(gtab_hbm.at[gidx_v], grows_v, sem_g)
        ca.wait()
        cg.wait()
        pltpu.sync_copy(arows_v, aout_hbm.at[pl.ds(base, b_per_w)])
        pltpu.sync_copy(grows_v, gout_hbm.at[pl.ds(base, b_per_w)])

    return gather_k(artist_ids, genre_ids, artist_table, genre_table)


def _tc_mlp(a_emb, g_emb, y_col, wy_row, by_row, w1a, w1g, w1y, b1_row, w2, b2_row):
    B, E = a_emb.shape
    HID = w1a.shape[1]
    OUT = w2.shape[1]
    BLK = 2048

    def mlp_k(a_ref, g_ref, y_ref, wy_ref, by_ref, w1a_ref, w1g_ref, w1y_ref,
              b1_ref, w2_ref, b2_ref, o_ref):
        y_emb = y_ref[...] * wy_ref[...] + by_ref[...]
        pre = (
            jnp.dot(a_ref[...], w1a_ref[...], preferred_element_type=jnp.float32)
            + jnp.dot(g_ref[...], w1g_ref[...], preferred_element_type=jnp.float32)
            + jnp.dot(y_emb, w1y_ref[...], preferred_element_type=jnp.float32)
            + b1_ref[...]
        )
        h = jnp.maximum(pre, 0.0)
        o_ref[...] = jnp.dot(h, w2_ref[...], preferred_element_type=jnp.float32) + b2_ref[...]

    return pl.pallas_call(
        mlp_k,
        grid=(B // BLK,),
        in_specs=[
            pl.BlockSpec((BLK, E), lambda i: (i, 0)),
            pl.BlockSpec((BLK, E), lambda i: (i, 0)),
            pl.BlockSpec((BLK, 1), lambda i: (i, 0)),
            pl.BlockSpec((1, E), lambda i: (0, 0)),
            pl.BlockSpec((1, E), lambda i: (0, 0)),
            pl.BlockSpec((E, HID), lambda i: (0, 0)),
            pl.BlockSpec((E, HID), lambda i: (0, 0)),
            pl.BlockSpec((E, HID), lambda i: (0, 0)),
            pl.BlockSpec((1, HID), lambda i: (0, 0)),
            pl.BlockSpec((HID, OUT), lambda i: (0, 0)),
            pl.BlockSpec((1, OUT), lambda i: (0, 0)),
        ],
        out_specs=pl.BlockSpec((BLK, OUT), lambda i: (i, 0)),
        out_shape=jax.ShapeDtypeStruct((B, OUT), jnp.float32),
    )(a_emb, g_emb, y_col, wy_row, by_row, w1a, w1g, w1y, b1_row, w2, b2_row)


def kernel(artist_ids, genre_ids, year_norms, artist_table, genre_table,
           Wy, by, W1, b1, W2, b2):
    E = artist_table.shape[1]
    a_emb, g_emb = _sc_gather(
        artist_ids.astype(jnp.int32),
        genre_ids.astype(jnp.int32),
        artist_table,
        genre_table,
    )
    y_col = year_norms[:, None]
    wy_row = Wy.T
    by_row = by[None, :]
    w1a = W1[:, :E].T
    w1g = W1[:, E:2 * E].T
    w1y = W1[:, 2 * E:3 * E].T
    b1_row = b1[None, :]
    w2 = W2.T
    b2_row = b2[None, :]
    return _tc_mlp(a_emb, g_emb, y_col, wy_row, by_row, w1a, w1g, w1y,
                   b1_row, w2, b2_row)
